# Initial kernel scaffold; baseline (speedup 1.0000x reference)
import jax
import jax.numpy as jnp
from jax.experimental import pallas as pl


def kernel(x, edge_index, edge_weight, Wxi, bxi, Whi, bhi, Wxf, bxf, Whf, bhf, Wxc, bxc, Whc, bhc, Wxo, bxo, Who, bho, w_ci, w_cf, w_co, b_i, b_f, b_c, b_o):
    raise NotImplementedError("write your pallas kernel here")


# speedup vs baseline: 23.3626x; 23.3626x over previous
import functools

import jax
import jax.numpy as jnp
from jax import lax
from jax.experimental import pallas as pl
from jax.experimental.pallas import tpu as pltpu
from jax.experimental.pallas import tpu_sc as plsc

N = 1000
NPAD = 1024
B = 8
C_IN = 32
H_DIM = 64
E = 16000
EPAD = 16384
NTILES = 16
EPT = EPAD // NTILES
NBURST = EPT // 16
ROWS_PT = NPAD // NTILES
FH = 128


def _rsqrt16(d):
    i = plsc.bitcast(d, jnp.int32)
    i = jnp.int32(0x5F3759DF) - lax.shift_right_logical(i, 1)
    y = plsc.bitcast(i, jnp.float32)
    for _ in range(3):
        y = y * (1.5 - 0.5 * d * y * y)
    return y


def _sc_body(x_hbm, row_hbm, rowd_hbm, col_hbm, w_hbm,
             t1_hbm, t2_hbm,
             row_v, rowd_v, col_v, w_v, dinv_v, xrows_v, gbuf, obuf, dv64,
             x_sp, acc1_sp, acc2_sp, deg_sp, dinv_sp):
    c = lax.axis_index("c")
    s = lax.axis_index("s")
    rbase = s * ROWS_PT

    pltpu.sync_copy(row_hbm.at[s], row_v)
    pltpu.sync_copy(rowd_hbm.at[s], rowd_v)
    pltpu.sync_copy(col_hbm.at[s], col_v)
    pltpu.sync_copy(w_hbm.at[s], w_v)
    pltpu.sync_copy(x_hbm.at[c, pl.ds(rbase, ROWS_PT), :], xrows_v)
    pltpu.sync_copy(xrows_v, x_sp.at[pl.ds(rbase, ROWS_PT), :])

    zero16 = jnp.zeros((16,), jnp.float32)

    def zrow(r, carry):
        for cc in range(FH // 16):
            obuf[r, cc * 16:(cc + 1) * 16] = zero16
        return carry

    lax.fori_loop(0, ROWS_PT, zrow, 0)
    pltpu.sync_copy(obuf, acc1_sp.at[pl.ds(rbase, ROWS_PT), :])
    pltpu.sync_copy(obuf, acc2_sp.at[pl.ds(rbase, ROWS_PT), :])
    for j in range(4):
        dv64[j * 16:(j + 1) * 16] = zero16
    pltpu.sync_copy(dv64, deg_sp.at[pl.ds(rbase, ROWS_PT)])
    plsc.subcore_barrier()

    for j in range(EPT // 128):
        pltpu.sync_copy(w_v.at[pl.ds(j * 128, 128)],
                        deg_sp.at[rowd_v.at[j]], add=True)
    plsc.subcore_barrier()

    pltpu.sync_copy(deg_sp.at[pl.ds(rbase, ROWS_PT)], dv64)
    for j in range(4):
        d = dv64[j * 16:(j + 1) * 16]
        pos = d > 0.0
        y = _rsqrt16(jnp.where(pos, d, 1.0))
        dv64[j * 16:(j + 1) * 16] = jnp.where(pos, y, 0.0)
    pltpu.sync_copy(dv64, dinv_sp.at[pl.ds(rbase, ROWS_PT)])
    plsc.subcore_barrier()
    pltpu.sync_copy(dinv_sp, dinv_v)

    def do_level(src_sp, dst_sp):
        def burst(b, carry):
            e0 = b * 16
            rows16 = row_v[pl.ds(e0, 16)]
            cols16 = col_v[pl.ds(e0, 16)]
            w16 = w_v[pl.ds(e0, 16)]
            dr = plsc.load_gather(dinv_v, [rows16])
            dc = plsc.load_gather(dinv_v, [cols16])
            wl = -(dr * w16 * dc)
            pltpu.sync_copy(src_sp.at[cols16], gbuf)
            for e in range(16):
                we = wl[e]
                for cc in range(FH // 16):
                    sl = slice(cc * 16, (cc + 1) * 16)
                    gbuf[e, sl] = gbuf[e, sl] * we
            pltpu.sync_copy(gbuf, dst_sp.at[rows16], add=True)
            return carry

        lax.fori_loop(0, NBURST, burst, 0)

    do_level(x_sp, acc1_sp)
    plsc.subcore_barrier()
    pltpu.sync_copy(acc1_sp.at[pl.ds(rbase, ROWS_PT), :], obuf)
    pltpu.sync_copy(obuf, t1_hbm.at[c, pl.ds(rbase, ROWS_PT), :])

    do_level(acc1_sp, acc2_sp)
    plsc.subcore_barrier()
    pltpu.sync_copy(acc2_sp.at[pl.ds(rbase, ROWS_PT), :], obuf)

    def t2row(r, carry):
        for cc in range(FH // 16):
            sl = slice(cc * 16, (cc + 1) * 16)
            obuf[r, sl] = 2.0 * obuf[r, sl] - xrows_v[r, sl]
        return carry

    lax.fori_loop(0, ROWS_PT, t2row, 0)
    pltpu.sync_copy(obuf, t2_hbm.at[c, pl.ds(rbase, ROWS_PT), :])


def _tc_body(z_ref, wi_ref, wc_ref, wo_ref, bi_ref, bc_ref, bo_ref, wco_ref,
             h_ref, c_ref):
    z = z_ref[0]
    gi = jnp.dot(z, wi_ref[...], preferred_element_type=jnp.float32) + bi_ref[...]
    gc = jnp.dot(z, wc_ref[...], preferred_element_type=jnp.float32) + bc_ref[...]
    go = jnp.dot(z, wo_ref[...], preferred_element_type=jnp.float32) + bo_ref[...]
    gate_i = jax.nn.sigmoid(gi)
    cand = jnp.tanh(gc)
    cell = gate_i * cand
    gate_o = jax.nn.sigmoid(go + wco_ref[...] * cell)
    h_ref[0] = gate_o * jnp.tanh(cell)
    c_ref[0] = cell


def kernel(x, edge_index, edge_weight, Wxi, bxi, Whi, bhi, Wxf, bxf, Whf, bhf,
           Wxc, bxc, Whc, bhc, Wxo, bxo, Who, bho, w_ci, w_cf, w_co,
           b_i, b_f, b_c, b_o):
    f32 = jnp.float32
    row = edge_index[0].astype(jnp.int32)
    col = edge_index[1].astype(jnp.int32)
    ew = edge_weight.astype(f32)

    pad_e = EPAD - E
    row_p = jnp.concatenate([row, jnp.zeros((pad_e,), jnp.int32)])
    col_p = jnp.concatenate([col, jnp.zeros((pad_e,), jnp.int32)])
    w_p = jnp.concatenate([ew, jnp.zeros((pad_e,), f32)])
    row_t = row_p.reshape(NTILES, EPT)
    rowd_t = row_p.reshape(NTILES, EPT // 128, 128)
    col_t = col_p.reshape(NTILES, EPT)
    w_t = w_p.reshape(NTILES, EPT)

    xt = x.transpose(1, 0, 2).reshape(N, B * C_IN)
    xt = jnp.concatenate([xt, jnp.zeros((NPAD - N, B * C_IN), f32)], axis=0)
    x_in = xt.reshape(NPAD, 2, FH).transpose(1, 0, 2)

    mesh = plsc.VectorSubcoreMesh(core_axis_name="c", subcore_axis_name="s")
    sc_fn = pl.kernel(
        _sc_body,
        out_type=[
            jax.ShapeDtypeStruct((2, NPAD, FH), f32),
            jax.ShapeDtypeStruct((2, NPAD, FH), f32),
        ],
        mesh=mesh,
        compiler_params=pltpu.CompilerParams(needs_layout_passes=False),
        scratch_types=[
            pltpu.VMEM((EPT,), jnp.int32),
            pltpu.VMEM((EPT // 128, 128), jnp.int32),
            pltpu.VMEM((EPT,), jnp.int32),
            pltpu.VMEM((EPT,), f32),
            pltpu.VMEM((NPAD,), f32),
            pltpu.VMEM((ROWS_PT, FH), f32),
            pltpu.VMEM((16, FH), f32),
            pltpu.VMEM((ROWS_PT, FH), f32),
            pltpu.VMEM((ROWS_PT,), f32),
            pltpu.VMEM_SHARED((NPAD, FH), f32),
            pltpu.VMEM_SHARED((NPAD, FH), f32),
            pltpu.VMEM_SHARED((NPAD, FH), f32),
            pltpu.VMEM_SHARED((NPAD,), f32),
            pltpu.VMEM_SHARED((NPAD,), f32),
        ],
    )
    t1s, t2s = sc_fn(x_in, row_t, rowd_t, col_t, w_t)

    def unlayout(ts):
        tt = ts.transpose(1, 0, 2).reshape(NPAD, B * C_IN)[:N]
        return tt.reshape(N, B, C_IN).transpose(1, 0, 2)

    t1 = unlayout(t1s)
    t2 = unlayout(t2s)
    z = jnp.concatenate([x, t1, t2], axis=2)

    def wcat(W):
        return jnp.concatenate([W[0], W[1], W[2]], axis=0)

    wi = wcat(Wxi)
    wc = wcat(Wxc)
    wo = wcat(Wxo)
    bi = (bxi + bhi + b_i[0]).reshape(1, H_DIM)
    bc = (bxc + bhc + b_c[0]).reshape(1, H_DIM)
    bo = (bxo + bho + b_o[0]).reshape(1, H_DIM)
    wco = w_co.reshape(1, H_DIM)

    kdim = 3 * C_IN
    h_out, c_out = pl.pallas_call(
        _tc_body,
        grid=(B,),
        in_specs=[
            pl.BlockSpec((1, N, kdim), lambda b: (b, 0, 0)),
            pl.BlockSpec((kdim, H_DIM), lambda b: (0, 0)),
            pl.BlockSpec((kdim, H_DIM), lambda b: (0, 0)),
            pl.BlockSpec((kdim, H_DIM), lambda b: (0, 0)),
            pl.BlockSpec((1, H_DIM), lambda b: (0, 0)),
            pl.BlockSpec((1, H_DIM), lambda b: (0, 0)),
            pl.BlockSpec((1, H_DIM), lambda b: (0, 0)),
            pl.BlockSpec((1, H_DIM), lambda b: (0, 0)),
        ],
        out_specs=[
            pl.BlockSpec((1, N, H_DIM), lambda b: (b, 0, 0)),
            pl.BlockSpec((1, N, H_DIM), lambda b: (b, 0, 0)),
        ],
        out_shape=[
            jax.ShapeDtypeStruct((B, N, H_DIM), f32),
            jax.ShapeDtypeStruct((B, N, H_DIM), f32),
        ],
    )(z, wi, wc, wo, bi, bc, bo, wco)

    return h_out, c_out


# speedup vs baseline: 25.0831x; 1.0736x over previous
import functools

import jax
import jax.numpy as jnp
from jax import lax
from jax.experimental import pallas as pl
from jax.experimental.pallas import tpu as pltpu
from jax.experimental.pallas import tpu_sc as plsc

N = 1000
NPAD = 1024
B = 8
C_IN = 32
H_DIM = 64
E = 16000
EPAD = 16384
NTILES = 16
EPT = EPAD // NTILES
NBURST = EPT // 16
ROWS_PT = NPAD // NTILES
FH = 128


def _rsqrt16(d):
    i = plsc.bitcast(d, jnp.int32)
    i = jnp.int32(0x5F3759DF) - lax.shift_right_logical(i, 1)
    y = plsc.bitcast(i, jnp.float32)
    for _ in range(3):
        y = y * (1.5 - 0.5 * d * y * y)
    return y


def _sc_body(x_hbm, row_hbm, rowd_hbm, col_hbm, cold_hbm, w_hbm,
             t1_hbm, t2_hbm,
             row_v, rowd_v, col_v, cold_v, w_v, wl_v, dinv_v, xrows_v, gbuf,
             obuf, dv64,
             x_sp, acc1_sp, acc2_sp, deg_sp, dinv_sp):
    c = lax.axis_index("c")
    s = lax.axis_index("s")
    rbase = s * ROWS_PT

    pltpu.sync_copy(row_hbm.at[s], row_v)
    pltpu.sync_copy(rowd_hbm.at[s], rowd_v)
    pltpu.sync_copy(col_hbm.at[s], col_v)
    pltpu.sync_copy(cold_hbm.at[s], cold_v)
    pltpu.sync_copy(w_hbm.at[s], w_v)
    pltpu.sync_copy(x_hbm.at[c, pl.ds(rbase, ROWS_PT), :], xrows_v)
    pltpu.sync_copy(xrows_v, x_sp.at[pl.ds(rbase, ROWS_PT), :])

    zero16 = jnp.zeros((16,), jnp.float32)

    def zrow(r, carry):
        for cc in range(FH // 16):
            obuf[r, cc * 16:(cc + 1) * 16] = zero16
        return carry

    lax.fori_loop(0, ROWS_PT, zrow, 0)
    pltpu.sync_copy(obuf, acc1_sp.at[pl.ds(rbase, ROWS_PT), :])
    pltpu.sync_copy(obuf, acc2_sp.at[pl.ds(rbase, ROWS_PT), :])
    for j in range(4):
        dv64[j * 16:(j + 1) * 16] = zero16
    pltpu.sync_copy(dv64, deg_sp.at[pl.ds(rbase, ROWS_PT)])
    plsc.subcore_barrier()

    for j in range(EPT // 128):
        pltpu.sync_copy(w_v.at[pl.ds(j * 128, 128)],
                        deg_sp.at[rowd_v.at[j]], add=True)
    plsc.subcore_barrier()

    pltpu.sync_copy(deg_sp.at[pl.ds(rbase, ROWS_PT)], dv64)
    for j in range(4):
        d = dv64[j * 16:(j + 1) * 16]
        pos = d > 0.0
        y = _rsqrt16(jnp.where(pos, d, 1.0))
        dv64[j * 16:(j + 1) * 16] = jnp.where(pos, y, 0.0)
    pltpu.sync_copy(dv64, dinv_sp.at[pl.ds(rbase, ROWS_PT)])
    plsc.subcore_barrier()
    pltpu.sync_copy(dinv_sp, dinv_v)

    def wlburst(b, carry):
        e0 = b * 16
        rows16 = row_v[pl.ds(e0, 16)]
        cols16 = col_v[pl.ds(e0, 16)]
        w16 = w_v[pl.ds(e0, 16)]
        dr = plsc.load_gather(dinv_v, [rows16])
        dc = plsc.load_gather(dinv_v, [cols16])
        wl_v[pl.ds(e0, 16)] = -(dr * w16 * dc)
        return carry

    lax.fori_loop(0, NBURST, wlburst, 0)

    def do_level(src_sp, dst_sp):
        for j in range(EPT // 128):
            pltpu.sync_copy(src_sp.at[cold_v.at[j]], gbuf)

            def grp(g, carry):
                e0 = g * 16
                wl16 = wl_v[pl.ds(j * 128 + e0, 16)]
                for e in range(16):
                    we = wl16[e]
                    r = e0 + e
                    for cc in range(FH // 16):
                        sl = slice(cc * 16, (cc + 1) * 16)
                        gbuf[r, sl] = gbuf[r, sl] * we
                return carry

            lax.fori_loop(0, 8, grp, 0)
            pltpu.sync_copy(gbuf, dst_sp.at[rowd_v.at[j]], add=True)

    do_level(x_sp, acc1_sp)
    plsc.subcore_barrier()
    pltpu.sync_copy(acc1_sp.at[pl.ds(rbase, ROWS_PT), :], obuf)
    pltpu.sync_copy(obuf, t1_hbm.at[c, pl.ds(rbase, ROWS_PT), :])

    do_level(acc1_sp, acc2_sp)
    plsc.subcore_barrier()
    pltpu.sync_copy(acc2_sp.at[pl.ds(rbase, ROWS_PT), :], obuf)

    def t2row(r, carry):
        for cc in range(FH // 16):
            sl = slice(cc * 16, (cc + 1) * 16)
            obuf[r, sl] = 2.0 * obuf[r, sl] - xrows_v[r, sl]
        return carry

    lax.fori_loop(0, ROWS_PT, t2row, 0)
    pltpu.sync_copy(obuf, t2_hbm.at[c, pl.ds(rbase, ROWS_PT), :])


def _tc_body(z_ref, wi_ref, wc_ref, wo_ref, bi_ref, bc_ref, bo_ref, wco_ref,
             h_ref, c_ref):
    z = z_ref[0]
    gi = jnp.dot(z, wi_ref[...], preferred_element_type=jnp.float32) + bi_ref[...]
    gc = jnp.dot(z, wc_ref[...], preferred_element_type=jnp.float32) + bc_ref[...]
    go = jnp.dot(z, wo_ref[...], preferred_element_type=jnp.float32) + bo_ref[...]
    gate_i = jax.nn.sigmoid(gi)
    cand = jnp.tanh(gc)
    cell = gate_i * cand
    gate_o = jax.nn.sigmoid(go + wco_ref[...] * cell)
    h_ref[0] = gate_o * jnp.tanh(cell)
    c_ref[0] = cell


def kernel(x, edge_index, edge_weight, Wxi, bxi, Whi, bhi, Wxf, bxf, Whf, bhf,
           Wxc, bxc, Whc, bhc, Wxo, bxo, Who, bho, w_ci, w_cf, w_co,
           b_i, b_f, b_c, b_o):
    f32 = jnp.float32
    row = edge_index[0].astype(jnp.int32)
    col = edge_index[1].astype(jnp.int32)
    ew = edge_weight.astype(f32)

    pad_e = EPAD - E
    row_p = jnp.concatenate([row, jnp.zeros((pad_e,), jnp.int32)])
    col_p = jnp.concatenate([col, jnp.zeros((pad_e,), jnp.int32)])
    w_p = jnp.concatenate([ew, jnp.zeros((pad_e,), f32)])
    row_t = row_p.reshape(NTILES, EPT)
    rowd_t = row_p.reshape(NTILES, EPT // 128, 128)
    col_t = col_p.reshape(NTILES, EPT)
    cold_t = col_p.reshape(NTILES, EPT // 128, 128)
    w_t = w_p.reshape(NTILES, EPT)

    xt = x.transpose(1, 0, 2).reshape(N, B * C_IN)
    xt = jnp.concatenate([xt, jnp.zeros((NPAD - N, B * C_IN), f32)], axis=0)
    x_in = xt.reshape(NPAD, 2, FH).transpose(1, 0, 2)

    mesh = plsc.VectorSubcoreMesh(core_axis_name="c", subcore_axis_name="s")
    sc_fn = pl.kernel(
        _sc_body,
        out_type=[
            jax.ShapeDtypeStruct((2, NPAD, FH), f32),
            jax.ShapeDtypeStruct((2, NPAD, FH), f32),
        ],
        mesh=mesh,
        compiler_params=pltpu.CompilerParams(needs_layout_passes=False),
        scratch_types=[
            pltpu.VMEM((EPT,), jnp.int32),
            pltpu.VMEM((EPT // 128, 128), jnp.int32),
            pltpu.VMEM((EPT,), jnp.int32),
            pltpu.VMEM((EPT // 128, 128), jnp.int32),
            pltpu.VMEM((EPT,), f32),
            pltpu.VMEM((EPT,), f32),
            pltpu.VMEM((NPAD,), f32),
            pltpu.VMEM((ROWS_PT, FH), f32),
            pltpu.VMEM((128, FH), f32),
            pltpu.VMEM((ROWS_PT, FH), f32),
            pltpu.VMEM((ROWS_PT,), f32),
            pltpu.VMEM_SHARED((NPAD, FH), f32),
            pltpu.VMEM_SHARED((NPAD, FH), f32),
            pltpu.VMEM_SHARED((NPAD, FH), f32),
            pltpu.VMEM_SHARED((NPAD,), f32),
            pltpu.VMEM_SHARED((NPAD,), f32),
        ],
    )
    t1s, t2s = sc_fn(x_in, row_t, rowd_t, col_t, cold_t, w_t)

    def unlayout(ts):
        tt = ts.transpose(1, 0, 2).reshape(NPAD, B * C_IN)[:N]
        return tt.reshape(N, B, C_IN).transpose(1, 0, 2)

    t1 = unlayout(t1s)
    t2 = unlayout(t2s)
    z = jnp.concatenate([x, t1, t2], axis=2)

    def wcat(W):
        return jnp.concatenate([W[0], W[1], W[2]], axis=0)

    wi = wcat(Wxi)
    wc = wcat(Wxc)
    wo = wcat(Wxo)
    bi = (bxi + bhi + b_i[0]).reshape(1, H_DIM)
    bc = (bxc + bhc + b_c[0]).reshape(1, H_DIM)
    bo = (bxo + bho + b_o[0]).reshape(1, H_DIM)
    wco = w_co.reshape(1, H_DIM)

    kdim = 3 * C_IN
    h_out, c_out = pl.pallas_call(
        _tc_body,
        grid=(B,),
        in_specs=[
            pl.BlockSpec((1, N, kdim), lambda b: (b, 0, 0)),
            pl.BlockSpec((kdim, H_DIM), lambda b: (0, 0)),
            pl.BlockSpec((kdim, H_DIM), lambda b: (0, 0)),
            pl.BlockSpec((kdim, H_DIM), lambda b: (0, 0)),
            pl.BlockSpec((1, H_DIM), lambda b: (0, 0)),
            pl.BlockSpec((1, H_DIM), lambda b: (0, 0)),
            pl.BlockSpec((1, H_DIM), lambda b: (0, 0)),
            pl.BlockSpec((1, H_DIM), lambda b: (0, 0)),
        ],
        out_specs=[
            pl.BlockSpec((1, N, H_DIM), lambda b: (b, 0, 0)),
            pl.BlockSpec((1, N, H_DIM), lambda b: (b, 0, 0)),
        ],
        out_shape=[
            jax.ShapeDtypeStruct((B, N, H_DIM), f32),
            jax.ShapeDtypeStruct((B, N, H_DIM), f32),
        ],
    )(z, wi, wc, wo, bi, bc, bo, wco)

    return h_out, c_out


# speedup vs baseline: 31.0594x; 1.2383x over previous
import functools

import jax
import jax.numpy as jnp
from jax import lax
from jax.experimental import pallas as pl
from jax.experimental.pallas import tpu as pltpu
from jax.experimental.pallas import tpu_sc as plsc

N = 1000
NPAD = 1024
B = 8
C_IN = 32
H_DIM = 64
E = 16000
EPAD = 16384
NTILES = 16
EPT = EPAD // NTILES
NBURST = EPT // 16
ROWS_PT = NPAD // NTILES
FH = 128


def _rsqrt16(d):
    i = plsc.bitcast(d, jnp.int32)
    i = jnp.int32(0x5F3759DF) - lax.shift_right_logical(i, 1)
    y = plsc.bitcast(i, jnp.float32)
    for _ in range(3):
        y = y * (1.5 - 0.5 * d * y * y)
    return y


def _sc_body(x_hbm, row_hbm, rowd_hbm, col_hbm, cold_hbm, w_hbm,
             t1_hbm, t2_hbm,
             row_v, rowd_v, col_v, cold_v, w_v, wl_v, dinv_v, xrows_v, gbuf,
             obuf, dv64,
             x_sp, acc1_sp, acc2_sp, deg_sp, dinv_sp):
    c = lax.axis_index("c")
    s = lax.axis_index("s")
    rbase = s * ROWS_PT

    pltpu.sync_copy(row_hbm.at[s], row_v)
    pltpu.sync_copy(rowd_hbm.at[s], rowd_v)
    pltpu.sync_copy(col_hbm.at[s], col_v)
    pltpu.sync_copy(cold_hbm.at[s], cold_v)
    pltpu.sync_copy(w_hbm.at[s], w_v)
    pltpu.sync_copy(x_hbm.at[c, pl.ds(rbase, ROWS_PT), :], xrows_v)
    pltpu.sync_copy(xrows_v, x_sp.at[pl.ds(rbase, ROWS_PT), :])

    zero16 = jnp.zeros((16,), jnp.float32)

    def zrow(r, carry):
        for cc in range(FH // 16):
            obuf[r, cc * 16:(cc + 1) * 16] = zero16
        return carry

    lax.fori_loop(0, ROWS_PT, zrow, 0)
    pltpu.sync_copy(obuf, acc1_sp.at[pl.ds(rbase, ROWS_PT), :])
    pltpu.sync_copy(obuf, acc2_sp.at[pl.ds(rbase, ROWS_PT), :])
    for j in range(4):
        dv64[j * 16:(j + 1) * 16] = zero16
    pltpu.sync_copy(dv64, deg_sp.at[pl.ds(rbase, ROWS_PT)])
    plsc.subcore_barrier()

    for j in range(EPT // 128):
        pltpu.sync_copy(w_v.at[pl.ds(j * 128, 128)],
                        deg_sp.at[rowd_v.at[j]], add=True)
    plsc.subcore_barrier()

    pltpu.sync_copy(deg_sp.at[pl.ds(rbase, ROWS_PT)], dv64)
    for j in range(4):
        d = dv64[j * 16:(j + 1) * 16]
        pos = d > 0.0
        y = _rsqrt16(jnp.where(pos, d, 1.0))
        dv64[j * 16:(j + 1) * 16] = jnp.where(pos, y, 0.0)
    pltpu.sync_copy(dv64, dinv_sp.at[pl.ds(rbase, ROWS_PT)])
    plsc.subcore_barrier()
    pltpu.sync_copy(dinv_sp, dinv_v)

    def wlburst(b, carry):
        e0 = b * 16
        rows16 = row_v[pl.ds(e0, 16)]
        cols16 = col_v[pl.ds(e0, 16)]
        w16 = w_v[pl.ds(e0, 16)]
        dr = plsc.load_gather(dinv_v, [rows16])
        dc = plsc.load_gather(dinv_v, [cols16])
        wl_v[pl.ds(e0, 16)] = -(dr * w16 * dc)
        return carry

    lax.fori_loop(0, NBURST, wlburst, 0)

    def do_level(src_sp, dst_sp):
        for j in range(EPT // 128):
            pltpu.sync_copy(src_sp.at[cold_v.at[j]], gbuf)

            def grp(g, carry):
                e0 = g * 16
                wl16 = wl_v[pl.ds(j * 128 + e0, 16)]
                for e in range(16):
                    we = wl16[e]
                    r = e0 + e
                    for cc in range(FH // 16):
                        sl = slice(cc * 16, (cc + 1) * 16)
                        gbuf[r, sl] = gbuf[r, sl] * we
                return carry

            lax.fori_loop(0, 8, grp, 0)
            pltpu.sync_copy(gbuf, dst_sp.at[rowd_v.at[j]], add=True)

    do_level(x_sp, acc1_sp)
    plsc.subcore_barrier()
    pltpu.sync_copy(acc1_sp.at[pl.ds(rbase, ROWS_PT), :], obuf)
    pltpu.sync_copy(obuf, t1_hbm.at[c, pl.ds(rbase, ROWS_PT), :])

    do_level(acc1_sp, acc2_sp)
    plsc.subcore_barrier()
    pltpu.sync_copy(acc2_sp.at[pl.ds(rbase, ROWS_PT), :], obuf)

    def t2row(r, carry):
        for cc in range(FH // 16):
            sl = slice(cc * 16, (cc + 1) * 16)
            obuf[r, sl] = 2.0 * obuf[r, sl] - xrows_v[r, sl]
        return carry

    lax.fori_loop(0, ROWS_PT, t2row, 0)
    pltpu.sync_copy(obuf, t2_hbm.at[c, pl.ds(rbase, ROWS_PT), :])


def _tc_body(x_ref, t1_ref, t2_ref, wi_ref, wc_ref, wo_ref, bi_ref, bc_ref,
             bo_ref, wco_ref, h_ref, c_ref):
    t1 = t1_ref[0]
    t2 = t2_ref[0]
    for k in range(4):
        sl = slice(k * C_IN, (k + 1) * C_IN)
        z = jnp.concatenate([x_ref[k], t1[:, sl], t2[:, sl]], axis=1)
        gi = jnp.dot(z, wi_ref[...], preferred_element_type=jnp.float32) + bi_ref[...]
        gc = jnp.dot(z, wc_ref[...], preferred_element_type=jnp.float32) + bc_ref[...]
        go = jnp.dot(z, wo_ref[...], preferred_element_type=jnp.float32) + bo_ref[...]
        gate_i = jax.nn.sigmoid(gi)
        cand = jnp.tanh(gc)
        cell = gate_i * cand
        gate_o = jax.nn.sigmoid(go + wco_ref[...] * cell)
        h_ref[k] = gate_o * jnp.tanh(cell)
        c_ref[k] = cell


def kernel(x, edge_index, edge_weight, Wxi, bxi, Whi, bhi, Wxf, bxf, Whf, bhf,
           Wxc, bxc, Whc, bhc, Wxo, bxo, Who, bho, w_ci, w_cf, w_co,
           b_i, b_f, b_c, b_o):
    f32 = jnp.float32
    row = edge_index[0].astype(jnp.int32)
    col = edge_index[1].astype(jnp.int32)
    ew = edge_weight.astype(f32)

    pad_e = EPAD - E
    row_p = jnp.concatenate([row, jnp.zeros((pad_e,), jnp.int32)])
    col_p = jnp.concatenate([col, jnp.zeros((pad_e,), jnp.int32)])
    w_p = jnp.concatenate([ew, jnp.zeros((pad_e,), f32)])
    row_t = row_p.reshape(NTILES, EPT)
    rowd_t = row_p.reshape(NTILES, EPT // 128, 128)
    col_t = col_p.reshape(NTILES, EPT)
    cold_t = col_p.reshape(NTILES, EPT // 128, 128)
    w_t = w_p.reshape(NTILES, EPT)

    xt = x.transpose(1, 0, 2).reshape(N, B * C_IN)
    xt = jnp.concatenate([xt, jnp.zeros((NPAD - N, B * C_IN), f32)], axis=0)
    x_in = xt.reshape(NPAD, 2, FH).transpose(1, 0, 2)

    mesh = plsc.VectorSubcoreMesh(core_axis_name="c", subcore_axis_name="s")
    sc_fn = pl.kernel(
        _sc_body,
        out_type=[
            jax.ShapeDtypeStruct((2, NPAD, FH), f32),
            jax.ShapeDtypeStruct((2, NPAD, FH), f32),
        ],
        mesh=mesh,
        compiler_params=pltpu.CompilerParams(needs_layout_passes=False),
        scratch_types=[
            pltpu.VMEM((EPT,), jnp.int32),
            pltpu.VMEM((EPT // 128, 128), jnp.int32),
            pltpu.VMEM((EPT,), jnp.int32),
            pltpu.VMEM((EPT // 128, 128), jnp.int32),
            pltpu.VMEM((EPT,), f32),
            pltpu.VMEM((EPT,), f32),
            pltpu.VMEM((NPAD,), f32),
            pltpu.VMEM((ROWS_PT, FH), f32),
            pltpu.VMEM((128, FH), f32),
            pltpu.VMEM((ROWS_PT, FH), f32),
            pltpu.VMEM((ROWS_PT,), f32),
            pltpu.VMEM_SHARED((NPAD, FH), f32),
            pltpu.VMEM_SHARED((NPAD, FH), f32),
            pltpu.VMEM_SHARED((NPAD, FH), f32),
            pltpu.VMEM_SHARED((NPAD,), f32),
            pltpu.VMEM_SHARED((NPAD,), f32),
        ],
    )
    t1s, t2s = sc_fn(x_in, row_t, rowd_t, col_t, cold_t, w_t)

    def wcat(W):
        return jnp.concatenate([W[0], W[1], W[2]], axis=0)

    wi = wcat(Wxi)
    wc = wcat(Wxc)
    wo = wcat(Wxo)
    bi = (bxi + bhi + b_i[0]).reshape(1, H_DIM)
    bc = (bxc + bhc + b_c[0]).reshape(1, H_DIM)
    bo = (bxo + bho + b_o[0]).reshape(1, H_DIM)
    wco = w_co.reshape(1, H_DIM)

    kdim = 3 * C_IN
    h_out, c_out = pl.pallas_call(
        _tc_body,
        grid=(2,),
        in_specs=[
            pl.BlockSpec((4, N, C_IN), lambda c: (c, 0, 0)),
            pl.BlockSpec((1, N, FH), lambda c: (c, 0, 0)),
            pl.BlockSpec((1, N, FH), lambda c: (c, 0, 0)),
            pl.BlockSpec((kdim, H_DIM), lambda c: (0, 0)),
            pl.BlockSpec((kdim, H_DIM), lambda c: (0, 0)),
            pl.BlockSpec((kdim, H_DIM), lambda c: (0, 0)),
            pl.BlockSpec((1, H_DIM), lambda c: (0, 0)),
            pl.BlockSpec((1, H_DIM), lambda c: (0, 0)),
            pl.BlockSpec((1, H_DIM), lambda c: (0, 0)),
            pl.BlockSpec((1, H_DIM), lambda c: (0, 0)),
        ],
        out_specs=[
            pl.BlockSpec((4, N, H_DIM), lambda c: (c, 0, 0)),
            pl.BlockSpec((4, N, H_DIM), lambda c: (c, 0, 0)),
        ],
        out_shape=[
            jax.ShapeDtypeStruct((B, N, H_DIM), f32),
            jax.ShapeDtypeStruct((B, N, H_DIM), f32),
        ],
    )(x, t1s, t2s, wi, wc, wo, bi, bc, bo, wco)

    return h_out, c_out


# speedup vs baseline: 35.5418x; 1.1443x over previous
import functools

import jax
import jax.numpy as jnp
from jax import lax
from jax.experimental import pallas as pl
from jax.experimental.pallas import tpu as pltpu
from jax.experimental.pallas import tpu_sc as plsc

N = 1000
NPAD = 1024
B = 8
C_IN = 32
H_DIM = 64
E = 16000
EPAD = 16384
NTILES = 16
EPT = EPAD // NTILES
NBURST = EPT // 16
ROWS_PT = NPAD // NTILES
FH = 128


def _rsqrt16(d):
    i = plsc.bitcast(d, jnp.int32)
    i = jnp.int32(0x5F3759DF) - lax.shift_right_logical(i, 1)
    y = plsc.bitcast(i, jnp.float32)
    for _ in range(3):
        y = y * (1.5 - 0.5 * d * y * y)
    return y


def _sc_body(x_hbm, row_hbm, rowd_hbm, col_hbm, cold_hbm, w_hbm,
             t1_hbm, t2_hbm,
             row_v, rowd_v, col_v, cold_v, w_v, wl_v, dinv_v, xrows_v, gbufA,
             gbufB, obuf, dv64, gsem, ssem,
             x_sp, acc1_sp, acc2_sp, deg_sp, dinv_sp):
    c = lax.axis_index("c")
    s = lax.axis_index("s")
    rbase = s * ROWS_PT

    pltpu.sync_copy(row_hbm.at[s], row_v)
    pltpu.sync_copy(rowd_hbm.at[s], rowd_v)
    pltpu.sync_copy(col_hbm.at[s], col_v)
    pltpu.sync_copy(cold_hbm.at[s], cold_v)
    pltpu.sync_copy(w_hbm.at[s], w_v)
    pltpu.sync_copy(x_hbm.at[c, pl.ds(rbase, ROWS_PT), :], xrows_v)
    pltpu.sync_copy(xrows_v, x_sp.at[pl.ds(rbase, ROWS_PT), :])

    zero16 = jnp.zeros((16,), jnp.float32)

    def zrow(r, carry):
        for cc in range(FH // 16):
            obuf[r, cc * 16:(cc + 1) * 16] = zero16
        return carry

    lax.fori_loop(0, ROWS_PT, zrow, 0)
    pltpu.sync_copy(obuf, acc1_sp.at[pl.ds(rbase, ROWS_PT), :])
    pltpu.sync_copy(obuf, acc2_sp.at[pl.ds(rbase, ROWS_PT), :])
    for j in range(4):
        dv64[j * 16:(j + 1) * 16] = zero16
    pltpu.sync_copy(dv64, deg_sp.at[pl.ds(rbase, ROWS_PT)])
    plsc.subcore_barrier()

    for j in range(EPT // 128):
        pltpu.sync_copy(w_v.at[pl.ds(j * 128, 128)],
                        deg_sp.at[rowd_v.at[j]], add=True)
    plsc.subcore_barrier()

    pltpu.sync_copy(deg_sp.at[pl.ds(rbase, ROWS_PT)], dv64)
    for j in range(4):
        d = dv64[j * 16:(j + 1) * 16]
        pos = d > 0.0
        y = _rsqrt16(jnp.where(pos, d, 1.0))
        dv64[j * 16:(j + 1) * 16] = jnp.where(pos, y, 0.0)
    pltpu.sync_copy(dv64, dinv_sp.at[pl.ds(rbase, ROWS_PT)])
    plsc.subcore_barrier()
    pltpu.sync_copy(dinv_sp, dinv_v)

    def wlburst(b, carry):
        e0 = b * 16
        rows16 = row_v[pl.ds(e0, 16)]
        cols16 = col_v[pl.ds(e0, 16)]
        w16 = w_v[pl.ds(e0, 16)]
        dr = plsc.load_gather(dinv_v, [rows16])
        dc = plsc.load_gather(dinv_v, [cols16])
        wl_v[pl.ds(e0, 16)] = -(dr * w16 * dc)
        return carry

    lax.fori_loop(0, NBURST, wlburst, 0)

    def do_level(src_sp, dst_sp):
        nb = EPT // 128
        bufs = (gbufA, gbufB)
        g_h = [None] * nb
        s_h = [None] * nb
        g_h[0] = pltpu.async_copy(src_sp.at[cold_v.at[0]], bufs[0], gsem)
        for j in range(nb):
            buf = bufs[j % 2]
            g_h[j].wait()
            if j + 1 < nb:
                if j >= 1:
                    s_h[j - 1].wait()
                g_h[j + 1] = pltpu.async_copy(
                    src_sp.at[cold_v.at[j + 1]], bufs[(j + 1) % 2], gsem)

            def grp(g, carry, _j=j, _buf=buf):
                e0 = g * 16
                wl16 = wl_v[pl.ds(_j * 128 + e0, 16)]
                for e in range(16):
                    we = wl16[e]
                    r = e0 + e
                    for cc in range(FH // 16):
                        sl = slice(cc * 16, (cc + 1) * 16)
                        _buf[r, sl] = _buf[r, sl] * we
                return carry

            lax.fori_loop(0, 8, grp, 0)
            s_h[j] = pltpu.async_copy(buf, dst_sp.at[rowd_v.at[j]], ssem,
                                      add=True)
        s_h[nb - 2].wait()
        s_h[nb - 1].wait()

    do_level(x_sp, acc1_sp)
    plsc.subcore_barrier()
    pltpu.sync_copy(acc1_sp.at[pl.ds(rbase, ROWS_PT), :], obuf)
    pltpu.sync_copy(obuf, t1_hbm.at[c, pl.ds(rbase, ROWS_PT), :])

    do_level(acc1_sp, acc2_sp)
    plsc.subcore_barrier()
    pltpu.sync_copy(acc2_sp.at[pl.ds(rbase, ROWS_PT), :], obuf)

    def t2row(r, carry):
        for cc in range(FH // 16):
            sl = slice(cc * 16, (cc + 1) * 16)
            obuf[r, sl] = 2.0 * obuf[r, sl] - xrows_v[r, sl]
        return carry

    lax.fori_loop(0, ROWS_PT, t2row, 0)
    pltpu.sync_copy(obuf, t2_hbm.at[c, pl.ds(rbase, ROWS_PT), :])


def _tc_body(x_ref, t1_ref, t2_ref, wi_ref, wc_ref, wo_ref, bi_ref, bc_ref,
             bo_ref, wco_ref, h_ref, c_ref):
    t1 = t1_ref[0]
    t2 = t2_ref[0]
    for k in range(4):
        sl = slice(k * C_IN, (k + 1) * C_IN)
        z = jnp.concatenate([x_ref[k], t1[:, sl], t2[:, sl]], axis=1)
        gi = jnp.dot(z, wi_ref[...], preferred_element_type=jnp.float32) + bi_ref[...]
        gc = jnp.dot(z, wc_ref[...], preferred_element_type=jnp.float32) + bc_ref[...]
        go = jnp.dot(z, wo_ref[...], preferred_element_type=jnp.float32) + bo_ref[...]
        gate_i = jax.nn.sigmoid(gi)
        cand = jnp.tanh(gc)
        cell = gate_i * cand
        gate_o = jax.nn.sigmoid(go + wco_ref[...] * cell)
        h_ref[k] = gate_o * jnp.tanh(cell)
        c_ref[k] = cell


def kernel(x, edge_index, edge_weight, Wxi, bxi, Whi, bhi, Wxf, bxf, Whf, bhf,
           Wxc, bxc, Whc, bhc, Wxo, bxo, Who, bho, w_ci, w_cf, w_co,
           b_i, b_f, b_c, b_o):
    f32 = jnp.float32
    row = edge_index[0].astype(jnp.int32)
    col = edge_index[1].astype(jnp.int32)
    ew = edge_weight.astype(f32)

    pad_e = EPAD - E
    row_p = jnp.concatenate([row, jnp.zeros((pad_e,), jnp.int32)])
    col_p = jnp.concatenate([col, jnp.zeros((pad_e,), jnp.int32)])
    w_p = jnp.concatenate([ew, jnp.zeros((pad_e,), f32)])
    row_t = row_p.reshape(NTILES, EPT)
    rowd_t = row_p.reshape(NTILES, EPT // 128, 128)
    col_t = col_p.reshape(NTILES, EPT)
    cold_t = col_p.reshape(NTILES, EPT // 128, 128)
    w_t = w_p.reshape(NTILES, EPT)

    xt = x.transpose(1, 0, 2).reshape(N, B * C_IN)
    xt = jnp.concatenate([xt, jnp.zeros((NPAD - N, B * C_IN), f32)], axis=0)
    x_in = xt.reshape(NPAD, 2, FH).transpose(1, 0, 2)

    mesh = plsc.VectorSubcoreMesh(core_axis_name="c", subcore_axis_name="s")
    sc_fn = pl.kernel(
        _sc_body,
        out_type=[
            jax.ShapeDtypeStruct((2, NPAD, FH), f32),
            jax.ShapeDtypeStruct((2, NPAD, FH), f32),
        ],
        mesh=mesh,
        compiler_params=pltpu.CompilerParams(needs_layout_passes=False),
        scratch_types=[
            pltpu.VMEM((EPT,), jnp.int32),
            pltpu.VMEM((EPT // 128, 128), jnp.int32),
            pltpu.VMEM((EPT,), jnp.int32),
            pltpu.VMEM((EPT // 128, 128), jnp.int32),
            pltpu.VMEM((EPT,), f32),
            pltpu.VMEM((EPT,), f32),
            pltpu.VMEM((NPAD,), f32),
            pltpu.VMEM((ROWS_PT, FH), f32),
            pltpu.VMEM((128, FH), f32),
            pltpu.VMEM((128, FH), f32),
            pltpu.VMEM((ROWS_PT, FH), f32),
            pltpu.VMEM((ROWS_PT,), f32),
            pltpu.SemaphoreType.DMA,
            pltpu.SemaphoreType.DMA,
            pltpu.VMEM_SHARED((NPAD, FH), f32),
            pltpu.VMEM_SHARED((NPAD, FH), f32),
            pltpu.VMEM_SHARED((NPAD, FH), f32),
            pltpu.VMEM_SHARED((NPAD,), f32),
            pltpu.VMEM_SHARED((NPAD,), f32),
        ],
    )
    t1s, t2s = sc_fn(x_in, row_t, rowd_t, col_t, cold_t, w_t)

    def wcat(W):
        return jnp.concatenate([W[0], W[1], W[2]], axis=0)

    wi = wcat(Wxi)
    wc = wcat(Wxc)
    wo = wcat(Wxo)
    bi = (bxi + bhi + b_i[0]).reshape(1, H_DIM)
    bc = (bxc + bhc + b_c[0]).reshape(1, H_DIM)
    bo = (bxo + bho + b_o[0]).reshape(1, H_DIM)
    wco = w_co.reshape(1, H_DIM)

    kdim = 3 * C_IN
    h_out, c_out = pl.pallas_call(
        _tc_body,
        grid=(2,),
        in_specs=[
            pl.BlockSpec((4, N, C_IN), lambda c: (c, 0, 0)),
            pl.BlockSpec((1, N, FH), lambda c: (c, 0, 0)),
            pl.BlockSpec((1, N, FH), lambda c: (c, 0, 0)),
            pl.BlockSpec((kdim, H_DIM), lambda c: (0, 0)),
            pl.BlockSpec((kdim, H_DIM), lambda c: (0, 0)),
            pl.BlockSpec((kdim, H_DIM), lambda c: (0, 0)),
            pl.BlockSpec((1, H_DIM), lambda c: (0, 0)),
            pl.BlockSpec((1, H_DIM), lambda c: (0, 0)),
            pl.BlockSpec((1, H_DIM), lambda c: (0, 0)),
            pl.BlockSpec((1, H_DIM), lambda c: (0, 0)),
        ],
        out_specs=[
            pl.BlockSpec((4, N, H_DIM), lambda c: (c, 0, 0)),
            pl.BlockSpec((4, N, H_DIM), lambda c: (c, 0, 0)),
        ],
        out_shape=[
            jax.ShapeDtypeStruct((B, N, H_DIM), f32),
            jax.ShapeDtypeStruct((B, N, H_DIM), f32),
        ],
    )(x, t1s, t2s, wi, wc, wo, bi, bc, bo, wco)

    return h_out, c_out
